# baseline (speedup 1.0000x reference)
import functools

import jax
import jax.numpy as jnp
from jax import lax
from jax.experimental import pallas as pl
from jax.experimental.pallas import tpu as pltpu
from jax.experimental.pallas import tpu_sc as plsc

N_CB = 4
K_CODES = 1024
D_EMB = 64
TILE_N = 2048
N_TOK = 4 * 4 * 32 * 32 * 4 // 4
ROWS = N_CB * N_TOK
NW = 32
CH = 512


def _tc_body(n_tokens, z_ref, embt2_ref, esq_ref,
             dist_ref, idx_ref, ppl_ref, counts, pacc):
    c = pl.program_id(0)
    b = pl.program_id(1)
    i = pl.program_id(2)
    nb = pl.num_programs(1)
    ni = pl.num_programs(2)

    z_blk = z_ref[0, 0]
    z_t = z_blk.T
    z_bf = z_t.astype(jnp.bfloat16)

    cross2 = jax.lax.dot_general(
        z_bf, embt2_ref[0], (((1,), (0,)), ((), ())),
        preferred_element_type=jnp.float32)
    z_sq = jnp.sum(z_t ** 2, axis=1, keepdims=True)
    dist = z_sq + cross2 + esq_ref[0]
    dist_ref[0] = dist

    dmin = jnp.min(dist, axis=1, keepdims=True)
    iota_nk = jax.lax.broadcasted_iota(jnp.int32, (TILE_N, K_CODES), 1)
    idx = jnp.min(jnp.where(dist == dmin, iota_nk, K_CODES), axis=1)
    idx_ref[0, 0, :] = idx + c * K_CODES

    oh_t = (jax.lax.broadcasted_iota(jnp.int32, (K_CODES, TILE_N), 0)
            == idx[None, :]).astype(jnp.float32)
    cnt = jnp.sum(oh_t, axis=1, keepdims=True)
    first = (b == 0) & (i == 0)
    counts[...] = jnp.where(first, cnt, counts[...] + cnt)

    @pl.when((c == 0) & first)
    def _init():
        pacc[0] = 0.0

    @pl.when((b == nb - 1) & (i == ni - 1))
    def _finish_codebook():
        avg = counts[...] * (1.0 / n_tokens)
        ent = jnp.sum(avg * jnp.log(avg + 1e-10))
        pacc[0] += jnp.exp(-ent)

        @pl.when(c == N_CB - 1)
        def _store_ppl():
            ppl_ref[...] = jnp.full((1, 1), pacc[0] * (1.0 / N_CB), jnp.float32)


_SC_MESH = plsc.VectorSubcoreMesh(core_axis_name="c", subcore_axis_name="s")


@functools.partial(
    pl.kernel, mesh=_SC_MESH,
    out_type=jax.ShapeDtypeStruct((ROWS, 128), jnp.float32),
    scratch_types=[
        pltpu.VMEM((CH,), jnp.int32),
        pltpu.VMEM((CH, 128), jnp.float32),
        pltpu.SemaphoreType.DMA,
    ],
)
def _sc_gather(table_hbm, idx_hbm, out_hbm, idx_v, rows_v, sem):
    wid = lax.axis_index("s") * 2 + lax.axis_index("c")
    rows_per_w = ROWS // NW
    base = wid * rows_per_w
    for j in range(rows_per_w // CH):
        off = base + j * CH
        pltpu.sync_copy(idx_hbm.at[pl.ds(off, CH)], idx_v)
        pltpu.async_copy(table_hbm.at[idx_v], rows_v, sem).wait()
        pltpu.sync_copy(rows_v, out_hbm.at[pl.ds(off, CH)])


def kernel(z, embeddings):
    B, C, t, h, w = z.shape
    nc, K, d = embeddings.shape
    T = t * h * w
    N = B * T
    nt = T // TILE_N

    zr = z.reshape(B, nc, d, T)
    embt = jnp.swapaxes(embeddings, 1, 2)
    embt2_bf = (embt * (-2.0)).astype(jnp.bfloat16)
    e_sq = jnp.transpose(
        jnp.sum(embeddings ** 2, axis=2, keepdims=True), (0, 2, 1))

    grid = (nc, B, nt)
    dist, idx3, ppl = pl.pallas_call(
        functools.partial(_tc_body, N),
        grid=grid,
        in_specs=[
            pl.BlockSpec((1, 1, d, TILE_N), lambda c, b, i: (b, c, 0, i)),
            pl.BlockSpec((1, d, K), lambda c, b, i: (c, 0, 0)),
            pl.BlockSpec((1, 1, K), lambda c, b, i: (c, 0, 0)),
        ],
        out_specs=[
            pl.BlockSpec((1, TILE_N, K), lambda c, b, i, _nt=nt: (c, b * _nt + i, 0)),
            pl.BlockSpec((1, 1, TILE_N),
                         lambda c, b, i, _nb=B, _nt=nt: ((c * _nb + b) * _nt + i, 0, 0)),
            pl.BlockSpec((1, 1), lambda c, b, i: (0, 0)),
        ],
        out_shape=[
            jax.ShapeDtypeStruct((nc, N, K), jnp.float32),
            jax.ShapeDtypeStruct((nc * B * nt, 1, TILE_N), jnp.int32),
            jax.ShapeDtypeStruct((1, 1), jnp.float32),
        ],
        scratch_shapes=[
            pltpu.VMEM((K, 1), jnp.float32),
            pltpu.SMEM((1,), jnp.float32),
        ],
        compiler_params=pltpu.CompilerParams(
            dimension_semantics=("arbitrary", "arbitrary", "arbitrary"),
        ),
    )(zr, embt2_bf, e_sq)

    table = jnp.pad(embeddings.reshape(nc * K, d), ((0, 0), (0, 128 - d)))
    idx_flat = idx3.reshape(ROWS)
    quant_rows = _sc_gather(table, idx_flat)[:, :d]
    quant_cm = jnp.transpose(
        quant_rows.reshape(nc, B, T, d), (1, 0, 3, 2)).reshape(B, C, t, h, w)
    quantized_st = z + (quant_cm - z)
    return quantized_st, dist, ppl[0, 0]


# speedup vs baseline: 1.2677x; 1.2677x over previous
import functools

import jax
import jax.numpy as jnp
from jax.experimental import pallas as pl
from jax.experimental.pallas import tpu as pltpu

N_CB = 4
K_CODES = 1024
D_EMB = 64
TILE_N = 2048


def _body(n_tokens, z_ref, embt2_ref, embt_ref, esq_ref,
          dist_ref, qst_ref, ppl_ref, counts, pacc):
    c = pl.program_id(0)
    b = pl.program_id(1)
    i = pl.program_id(2)
    nb = pl.num_programs(1)
    ni = pl.num_programs(2)

    z_blk = z_ref[0, 0]
    z_t = z_blk.T
    z_bf = z_t.astype(jnp.bfloat16)

    cross2 = jax.lax.dot_general(
        z_bf, embt2_ref[0], (((1,), (0,)), ((), ())),
        preferred_element_type=jnp.float32)
    z_sq = jnp.sum(z_t ** 2, axis=1, keepdims=True)
    dist = z_sq + cross2 + esq_ref[0]
    dist_ref[0] = dist

    dmin = jnp.min(dist, axis=1, keepdims=True)
    iota_nk = jax.lax.broadcasted_iota(jnp.int32, (TILE_N, K_CODES), 1)
    idx = jnp.min(jnp.where(dist == dmin, iota_nk, K_CODES), axis=1)
    oh_t = (jax.lax.broadcasted_iota(jnp.int32, (K_CODES, TILE_N), 0)
            == idx[None, :]).astype(jnp.float32)
    quant = jax.lax.dot_general(
        embt_ref[0], oh_t, (((1,), (0,)), ((), ())),
        preferred_element_type=jnp.float32)
    qst_ref[0, 0] = z_blk + (quant - z_blk)

    cnt = jnp.sum(oh_t, axis=1, keepdims=True)
    first = (b == 0) & (i == 0)
    counts[...] = jnp.where(first, cnt, counts[...] + cnt)

    @pl.when((c == 0) & first)
    def _init():
        pacc[0] = 0.0

    @pl.when((b == nb - 1) & (i == ni - 1))
    def _finish_codebook():
        avg = counts[...] * (1.0 / n_tokens)
        ent = jnp.sum(avg * jnp.log(avg + 1e-10))
        pacc[0] += jnp.exp(-ent)

        @pl.when(c == N_CB - 1)
        def _store_ppl():
            ppl_ref[...] = jnp.full((1, 1), pacc[0] * (1.0 / N_CB), jnp.float32)


def kernel(z, embeddings):
    B, C, t, h, w = z.shape
    nc, K, d = embeddings.shape
    T = t * h * w
    N = B * T
    nt = T // TILE_N

    zr = z.reshape(B, nc, d, T)
    embt = jnp.swapaxes(embeddings, 1, 2)
    embt2_bf = (embt * (-2.0)).astype(jnp.bfloat16)
    embt_bf = embt.astype(jnp.bfloat16)
    e_sq = jnp.transpose(
        jnp.sum(embeddings ** 2, axis=2, keepdims=True), (0, 2, 1))

    grid = (nc, B, nt)
    dist, qst, ppl = pl.pallas_call(
        functools.partial(_body, N),
        grid=grid,
        in_specs=[
            pl.BlockSpec((1, 1, d, TILE_N), lambda c, b, i: (b, c, 0, i)),
            pl.BlockSpec((1, d, K), lambda c, b, i: (c, 0, 0)),
            pl.BlockSpec((1, d, K), lambda c, b, i: (c, 0, 0)),
            pl.BlockSpec((1, 1, K), lambda c, b, i: (c, 0, 0)),
        ],
        out_specs=[
            pl.BlockSpec((1, TILE_N, K), lambda c, b, i, _nt=nt: (c, b * _nt + i, 0)),
            pl.BlockSpec((1, 1, d, TILE_N), lambda c, b, i: (b, c, 0, i)),
            pl.BlockSpec((1, 1), lambda c, b, i: (0, 0)),
        ],
        out_shape=[
            jax.ShapeDtypeStruct((nc, N, K), jnp.float32),
            jax.ShapeDtypeStruct((B, nc, d, T), jnp.float32),
            jax.ShapeDtypeStruct((1, 1), jnp.float32),
        ],
        scratch_shapes=[
            pltpu.VMEM((K, 1), jnp.float32),
            pltpu.SMEM((1,), jnp.float32),
        ],
        compiler_params=pltpu.CompilerParams(
            dimension_semantics=("arbitrary", "arbitrary", "arbitrary"),
        ),
    )(zr, embt2_bf, embt_bf, e_sq)

    quantized_st = qst.reshape(B, C, t, h, w)
    return quantized_st, dist, ppl[0, 0]


# speedup vs baseline: 1.2759x; 1.0064x over previous
import functools

import jax
import jax.numpy as jnp
from jax.experimental import pallas as pl
from jax.experimental.pallas import tpu as pltpu

N_CB = 4
K_CODES = 1024
D_EMB = 64
TILE_N = 4096


def _body(n_tokens, z_ref, embt2_ref, embt_ref, esq_ref,
          dist_ref, qst_ref, ppl_ref, counts, pacc):
    c = pl.program_id(0)
    b = pl.program_id(1)
    i = pl.program_id(2)
    nb = pl.num_programs(1)
    ni = pl.num_programs(2)

    z_blk = z_ref[0, 0]
    z_t = z_blk.T
    z_bf = z_t.astype(jnp.bfloat16)

    cross2 = jax.lax.dot_general(
        z_bf, embt2_ref[0], (((1,), (0,)), ((), ())),
        preferred_element_type=jnp.float32)
    z_sq = jnp.sum(z_t ** 2, axis=1, keepdims=True)
    dist = z_sq + cross2 + esq_ref[0]
    dist_ref[0] = dist

    dmin = jnp.min(dist, axis=1, keepdims=True)
    iota_nk = jax.lax.broadcasted_iota(jnp.int32, (TILE_N, K_CODES), 1)
    idx = jnp.min(jnp.where(dist == dmin, iota_nk, K_CODES), axis=1)
    oh_t = (jax.lax.broadcasted_iota(jnp.int32, (K_CODES, TILE_N), 0)
            == idx[None, :]).astype(jnp.float32)
    quant = jax.lax.dot_general(
        embt_ref[0], oh_t, (((1,), (0,)), ((), ())),
        preferred_element_type=jnp.float32)
    qst_ref[0, 0] = z_blk + (quant - z_blk)

    cnt = jnp.sum(oh_t, axis=1, keepdims=True)
    first = (b == 0) & (i == 0)
    counts[...] = jnp.where(first, cnt, counts[...] + cnt)

    @pl.when((c == 0) & first)
    def _init():
        pacc[0] = 0.0

    @pl.when((b == nb - 1) & (i == ni - 1))
    def _finish_codebook():
        avg = counts[...] * (1.0 / n_tokens)
        ent = jnp.sum(avg * jnp.log(avg + 1e-10))
        pacc[0] += jnp.exp(-ent)

        @pl.when(c == N_CB - 1)
        def _store_ppl():
            ppl_ref[...] = jnp.full((1, 1), pacc[0] * (1.0 / N_CB), jnp.float32)


def kernel(z, embeddings):
    B, C, t, h, w = z.shape
    nc, K, d = embeddings.shape
    T = t * h * w
    N = B * T
    nt = T // TILE_N

    zr = z.reshape(B, nc, d, T)
    embt = jnp.swapaxes(embeddings, 1, 2)
    embt2_bf = (embt * (-2.0)).astype(jnp.bfloat16)
    embt_bf = embt.astype(jnp.bfloat16)
    e_sq = jnp.transpose(
        jnp.sum(embeddings ** 2, axis=2, keepdims=True), (0, 2, 1))

    grid = (nc, B, nt)
    dist, qst, ppl = pl.pallas_call(
        functools.partial(_body, N),
        grid=grid,
        in_specs=[
            pl.BlockSpec((1, 1, d, TILE_N), lambda c, b, i: (b, c, 0, i)),
            pl.BlockSpec((1, d, K), lambda c, b, i: (c, 0, 0)),
            pl.BlockSpec((1, d, K), lambda c, b, i: (c, 0, 0)),
            pl.BlockSpec((1, 1, K), lambda c, b, i: (c, 0, 0)),
        ],
        out_specs=[
            pl.BlockSpec((1, TILE_N, K), lambda c, b, i, _nt=nt: (c, b * _nt + i, 0)),
            pl.BlockSpec((1, 1, d, TILE_N), lambda c, b, i: (b, c, 0, i)),
            pl.BlockSpec((1, 1), lambda c, b, i: (0, 0)),
        ],
        out_shape=[
            jax.ShapeDtypeStruct((nc, N, K), jnp.float32),
            jax.ShapeDtypeStruct((B, nc, d, T), jnp.float32),
            jax.ShapeDtypeStruct((1, 1), jnp.float32),
        ],
        scratch_shapes=[
            pltpu.VMEM((K, 1), jnp.float32),
            pltpu.SMEM((1,), jnp.float32),
        ],
        compiler_params=pltpu.CompilerParams(
            dimension_semantics=("arbitrary", "arbitrary", "arbitrary"),
        ),
    )(zr, embt2_bf, embt_bf, e_sq)

    quantized_st = qst.reshape(B, C, t, h, w)
    return quantized_st, dist, ppl[0, 0]
